# baseline (speedup 1.0000x reference)
import functools

import jax
import jax.numpy as jnp
from jax import lax
from jax.experimental import pallas as pl
from jax.experimental.pallas import tpu as pltpu
from jax.experimental.pallas import tpu_sc as plsc

N, D, E, F, C = 10000, 128, 320000, 64, 8

NC, NS = 2, 16
NW = NC * NS
K = 80
K2 = 40
NB32 = E // NW // K
NB32B = E // NW // K2
NB16 = E // NS // K
NPAD = 10240
ZCH = 64
RPS = NPAD // NS
BR = 1280

_MESH = dict(core_axis_name="c", subcore_axis_name="s", num_cores=NC,
             num_subcores=NS)
_SC_PARAMS = pltpu.CompilerParams(use_tc_tiling_on_sc=False)
NBUF = 5


def _pipelined_edges(h_ref, src_v, dst_v, rows_v, sems, acc, nb, ones_v,
                     dacc):
    nbuf = len(sems)
    for b in range(nbuf):
        pltpu.async_copy(h_ref.at[src_v.at[b]], rows_v.at[b], sems[b])

    def outer(i, _):
        for b in range(nbuf):
            j = i * nbuf + b
            pltpu.make_async_copy(
                h_ref.at[src_v.at[j]], rows_v.at[b], sems[b]).wait()
            pltpu.sync_copy(rows_v.at[b], acc.at[dst_v.at[j]], add=True)
            if ones_v is not None:
                pltpu.sync_copy(ones_v, dacc.at[dst_v.at[j]], add=True)
            nj = j + nbuf

            @pl.when(nj < nb)
            def _():
                pltpu.async_copy(h_ref.at[src_v.at[nj]], rows_v.at[b],
                                 sems[b])
        return 0

    lax.fori_loop(0, nb // nbuf, outer, 0)


def _mul_rows(buf, other, rows, w):
    c16 = w // 16

    def body(i, _):
        r = i // c16
        cc = pl.ds((i % c16) * 16, 16)
        buf[r, cc] = buf[r, cc] * other[r, cc]
        return 0

    lax.fori_loop(0, rows * c16, body, 0)


def _zero_rows(buf, rows, w):
    assert w % 16 == 0, w
    c16 = w // 16
    zv = jnp.zeros((16,), jnp.float32)

    def body(i, _):
        buf[i // c16, pl.ds((i % c16) * 16, 16)] = zv
        return 0

    lax.fori_loop(0, rows * c16, body, 0)


def _make_prop_split(with_deg):
    W = 64
    nbuf = NBUF
    mesh = plsc.VectorSubcoreMesh(**_MESH)
    if with_deg:
        out_type = (jax.ShapeDtypeStruct((NC, NPAD, W), jnp.float32),
                    jax.ShapeDtypeStruct((NPAD,), jnp.float32))
    else:
        out_type = jax.ShapeDtypeStruct((NC, NPAD, W), jnp.float32)

    scratch = [
        pltpu.VMEM((NB16, K), jnp.int32),
        pltpu.VMEM((NB16, K), jnp.int32),
        pltpu.VMEM((nbuf, K, W), jnp.float32),
        pltpu.VMEM((ZCH, W), jnp.float32),
        pltpu.VMEM_SHARED((NPAD, W), jnp.float32),
    ] + [pltpu.SemaphoreType.DMA] * nbuf
    if with_deg:
        scratch.append(pltpu.VMEM((K,), jnp.float32))
        scratch.append(pltpu.VMEM((RPS,), jnp.float32))
        scratch.append(pltpu.VMEM_SHARED((NPAD,), jnp.float32))

    def body(ha_hbm, hb_hbm, src_hbm, dst_hbm, *rest):
        if with_deg:
            out_hbm, deg_hbm, src_v, dst_v, rows_v, zbuf, acc = rest[:7]
            sems = rest[7:7 + nbuf]
            ones_v, zrow, dacc = rest[7 + nbuf:]
        else:
            out_hbm, src_v, dst_v, rows_v, zbuf, acc = rest[:6]
            sems = rest[6:6 + nbuf]
            deg_hbm = ones_v = zrow = dacc = None
        cid = lax.axis_index("c")
        sid = lax.axis_index("s")
        lo = sid * RPS

        _zero_rows(zbuf, ZCH, W)
        for t in range(RPS // ZCH):
            pltpu.sync_copy(zbuf, acc.at[pl.ds(lo + t * ZCH, ZCH)])
        if with_deg:
            def dz(i, _):
                zrow[pl.ds(i * 16, 16)] = jnp.zeros((16,), jnp.float32)
                return 0
            lax.fori_loop(0, RPS // 16, dz, 0)

            def of(i, _):
                ones_v[pl.ds(i * 16, 16)] = jnp.ones((16,), jnp.float32)
                return 0
            lax.fori_loop(0, K // 16, of, 0)

            @pl.when(cid == 0)
            def _():
                pltpu.sync_copy(zrow, dacc.at[pl.ds(lo, RPS)])

        pltpu.sync_copy(src_hbm.at[sid], src_v)
        pltpu.sync_copy(dst_hbm.at[sid], dst_v)

        plsc.subcore_barrier()

        def edge_loop(h_ref, do_deg):
            _pipelined_edges(h_ref, src_v, dst_v, rows_v, sems, acc, NB16,
                             ones_v if do_deg else None, dacc)

        @pl.when(cid == 0)
        def _():
            edge_loop(ha_hbm, with_deg)

        @pl.when(cid == 1)
        def _():
            edge_loop(hb_hbm, False)

        plsc.subcore_barrier()

        pltpu.sync_copy(acc.at[pl.ds(lo, RPS)],
                        out_hbm.at[cid, pl.ds(lo, RPS)])
        if with_deg:
            @pl.when(cid == 0)
            def _():
                pltpu.sync_copy(dacc.at[pl.ds(lo, RPS)],
                                deg_hbm.at[pl.ds(lo, RPS)])

    return pl.kernel(body, out_type=out_type, mesh=mesh,
                     scratch_types=scratch, compiler_params=_SC_PARAMS)


def _make_prop_partial(W):
    nbuf = 25 if W <= 16 else 10
    kk = K if W <= 16 else K2
    nb = NB32 if W <= 16 else NB32B
    mesh = plsc.VectorSubcoreMesh(**_MESH)
    scratch = [
        pltpu.VMEM((nb, kk), jnp.int32),
        pltpu.VMEM((nb, kk), jnp.int32),
        pltpu.VMEM((nbuf, kk, W), jnp.float32),
        pltpu.VMEM((ZCH, W), jnp.float32),
        pltpu.VMEM_SHARED((NPAD, W), jnp.float32),
    ] + [pltpu.SemaphoreType.DMA] * nbuf

    def body(h_hbm, src_hbm, dst_hbm, out_hbm, src_v, dst_v, rows_v, zbuf,
             acc, *sems):
        cid = lax.axis_index("c")
        sid = lax.axis_index("s")
        w = cid * NS + sid

        _zero_rows(zbuf, ZCH, W)
        for t in range(RPS // ZCH):
            pltpu.sync_copy(zbuf, acc.at[pl.ds(sid * RPS + t * ZCH, ZCH)])

        pltpu.sync_copy(src_hbm.at[w], src_v)
        pltpu.sync_copy(dst_hbm.at[w], dst_v)

        plsc.subcore_barrier()

        _pipelined_edges(h_hbm, src_v, dst_v, rows_v, sems, acc, nb,
                         None, None)

        plsc.subcore_barrier()

        pltpu.sync_copy(acc.at[pl.ds(sid * RPS, RPS)],
                        out_hbm.at[cid, pl.ds(sid * RPS, RPS)])

    return pl.kernel(
        body,
        out_type=jax.ShapeDtypeStruct((NC, NPAD, W), jnp.float32),
        mesh=mesh,
        scratch_types=scratch,
        compiler_params=_SC_PARAMS,
    )


def _make_stage2():
    W = 16
    HALF = NPAD // NC
    OPS = HALF // NS
    mesh = plsc.VectorSubcoreMesh(**_MESH)
    out_type = (jax.ShapeDtypeStruct((NPAD, W), jnp.float32),
                jax.ShapeDtypeStruct((NC, NPAD, W), jnp.float32),
                jax.ShapeDtypeStruct((NC, NPAD, W), jnp.float32))
    scratch = [
        pltpu.VMEM((NB16, K), jnp.int32),
        pltpu.VMEM((NB16, K), jnp.int32),
        pltpu.VMEM((NBUF, K, W), jnp.float32),
        pltpu.VMEM((RPS, W), jnp.float32),
        pltpu.VMEM((RPS, W), jnp.float32),
        pltpu.VMEM((RPS, W), jnp.float32),
        pltpu.VMEM((RPS, W), jnp.float32),
        pltpu.VMEM((16,), jnp.float32),
        pltpu.VMEM_SHARED((NPAD, W), jnp.float32),
    ] + [pltpu.SemaphoreType.DMA] * NBUF

    def body(c_hbm, b_hbm, a_hbm, invd_hbm, b2_hbm, src_hbm, dst_hbm,
             out_hbm, s1_hbm, s2_hbm, src_v, dst_v, rows_v, zblk, work,
             addv, invd_v, b2v, acc, *sems):
        cid = lax.axis_index("c")
        sid = lax.axis_index("s")
        lo = sid * RPS

        _zero_rows(zblk, RPS, W)
        pltpu.sync_copy(zblk, acc.at[pl.ds(lo, RPS)])
        pltpu.sync_copy(src_hbm.at[sid], src_v)
        pltpu.sync_copy(dst_hbm.at[sid], dst_v)
        pltpu.sync_copy(invd_hbm.at[pl.ds(lo, RPS)], invd_v)
        pltpu.sync_copy(b2_hbm, b2v)

        def combine(add_src_hbm, dst_ref):
            pltpu.sync_copy(acc.at[pl.ds(lo, RPS)], work)
            pltpu.sync_copy(add_src_hbm.at[pl.ds(lo, RPS)], addv)

            def row(n, _):
                work[n, :] = addv[n, :] + work[n, :] * invd_v[n, :]
                return 0

            lax.fori_loop(0, RPS, row, 0)
            pltpu.sync_copy(work, dst_ref.at[cid, pl.ds(lo, RPS)])
            pltpu.sync_copy(zblk, acc.at[pl.ds(lo, RPS)])

        plsc.subcore_barrier()
        _pipelined_edges(c_hbm, src_v, dst_v, rows_v, sems, acc, NB16,
                         None, None)
        plsc.subcore_barrier()
        combine(b_hbm, s1_hbm)
        plsc.subcore_barrier()
        _pipelined_edges(s1_hbm.at[cid], src_v, dst_v, rows_v, sems, acc,
                         NB16, None, None)
        plsc.subcore_barrier()
        combine(a_hbm, s2_hbm)
        plsc.subcore_barrier()
        _pipelined_edges(s2_hbm.at[cid], src_v, dst_v, rows_v, sems, acc,
                         NB16, None, None)
        plsc.subcore_barrier()

        base = cid * HALF + sid * OPS
        pltpu.sync_copy(acc.at[pl.ds(base, OPS)], work.at[pl.ds(0, OPS)])
        pltpu.sync_copy(invd_hbm.at[pl.ds(base, OPS)],
                        invd_v.at[pl.ds(0, OPS)])
        b2r = b2v[...]

        def orow(n, _):
            work[n, :] = jnp.maximum(work[n, :] * invd_v[n, :] + b2r, 0.0)
            return 0

        lax.fori_loop(0, OPS, orow, 0)
        pltpu.sync_copy(work.at[pl.ds(0, OPS)], out_hbm.at[pl.ds(base, OPS)])

    return pl.kernel(body, out_type=out_type, mesh=mesh,
                     scratch_types=scratch,
                     compiler_params=pltpu.CompilerParams(
                         use_tc_tiling_on_sc=False,
                         needs_layout_passes=False))


@functools.lru_cache(maxsize=None)
def _get_prop(kind):
    if kind == "stage2":
        return _make_stage2()
    if kind == "split_deg":
        return _make_prop_split(True)
    if kind == "split":
        return _make_prop_split(False)
    return _make_prop_partial(kind)


def _prop128_deg(*a):
    return _get_prop("split_deg")(*a)


def _prop128(*a):
    return _get_prop("split")(*a)


def _prop64(*a):
    return _get_prop(64)(*a)


def _prop16(*a):
    return _get_prop(16)(*a)


def _stage2(*a):
    return _get_prop("stage2")(*a)


def _tc_call(body_fn, in_specs, out_specs, out_shapes):
    return pl.pallas_call(
        body_fn,
        grid=(NPAD // BR,),
        in_specs=in_specs,
        out_specs=out_specs,
        out_shape=out_shapes,
    )


def _spec3(w):
    return pl.BlockSpec((NC, BR, w), lambda i: (0, i, 0))


def _spec2(w):
    return pl.BlockSpec((BR, w), lambda i: (i, 0))


def _spec1():
    return pl.BlockSpec((BR, 1), lambda i: (i, 0))


def _full(shape):
    nd = len(shape)
    return pl.BlockSpec(shape, lambda i: (0,) * nd)


def _tc1_body(p_ref, dg_ref, w1a_ref, w1bc_ref, b1a_ref,
              z0_ref, z1_ref, h1a_ref, invd_ref):
    inv = 1.0 / jnp.maximum(dg_ref[...], 1.0)
    h1 = jnp.concatenate([p_ref[0], p_ref[1]], axis=1) * inv
    z = jnp.dot(h1, w1bc_ref[...], preferred_element_type=jnp.float32)
    z0_ref[...] = z[:, :F]
    z1_ref[...] = z[:, F:]
    h1a_ref[...] = (jnp.dot(h1, w1a_ref[...], preferred_element_type=jnp.float32)
                    + b1a_ref[...][None, :])
    invd_ref[...] = inv


def _tc2_body(r_ref, invd_ref, b1b_ref, u_ref, qb_ref):
    inv = invd_ref[...]
    u_ref[...] = r_ref[1] * inv
    qb_ref[...] = r_ref[0] * inv + b1b_ref[...][None, :]


def _tc2_body(r_ref, invd_ref, b1b_ref, u_ref, qb_ref):
    inv = invd_ref[...]
    u_ref[...] = r_ref[1] * inv
    qb_ref[...] = r_ref[0] * inv + b1b_ref[...][None, :]


def _tc3_body(r_ref, invd_ref, h1a_ref, qb_ref, b1c_ref, w2_ref, abc_ref):
    v = (r_ref[0] + r_ref[1]) * invd_ref[...] + b1c_ref[...][None, :]
    h = jax.nn.relu(jnp.concatenate([h1a_ref[...], qb_ref[...], v], axis=1))
    abc_ref[...] = jnp.dot(h, w2_ref[...], preferred_element_type=jnp.float32)


def _tc_addnorm_body(r_ref, invd_ref, add_ref, o_ref):
    o_ref[...] = add_ref[...] + (r_ref[0] + r_ref[1]) * invd_ref[...]


def _tc_final_body(r_ref, invd_ref, b2_ref, o_ref):
    t = (r_ref[0] + r_ref[1])[:, :C] * invd_ref[...]
    o_ref[...] = jax.nn.relu(t + b2_ref[...][None, :])


def kernel(x, edge_index, W1a, W1b, W1c, b1a, b1b, b1c, W2a, W2b, W2c, b2):
    src16 = edge_index[0].reshape(NS, NB16, K)
    dst16 = edge_index[1].reshape(NS, NB16, K)
    src32 = edge_index[0].reshape(NW, NB32, K)
    dst32 = edge_index[1].reshape(NW, NB32, K)
    src32b = edge_index[0].reshape(NW, NB32B, K2)
    dst32b = edge_index[1].reshape(NW, NB32B, K2)
    W1bc = jnp.concatenate([W1b, W1c], axis=1)
    zc = jnp.zeros((3 * F, 8), jnp.float32)
    W2abc = jnp.concatenate([W2a, zc, W2b, zc, W2c, zc], axis=1)

    p1, deg = _prop128_deg(x[:, :F], x[:, F:], src16, dst16)
    deg2 = deg.reshape(NPAD, 1)

    z0, z1, h1a, invd = _tc_call(
        _tc1_body,
        [_spec3(F), _spec1(), _full((D, F)), _full((D, D)), _full((F,))],
        [_spec2(F), _spec2(F), _spec2(F), _spec1()],
        [jax.ShapeDtypeStruct((NPAD, F), jnp.float32),
         jax.ShapeDtypeStruct((NPAD, F), jnp.float32),
         jax.ShapeDtypeStruct((NPAD, F), jnp.float32),
         jax.ShapeDtypeStruct((NPAD, 1), jnp.float32)],
    )(p1, deg2, W1a, W1bc, b1a)

    r2 = _prop128(z0, z1, src16, dst16)

    u, qb = _tc_call(
        _tc2_body,
        [_spec3(F), _spec1(), _full((F,))],
        [_spec2(F), _spec2(F)],
        [jax.ShapeDtypeStruct((NPAD, F), jnp.float32),
         jax.ShapeDtypeStruct((NPAD, F), jnp.float32)],
    )(r2, invd, b1b)

    r3 = _prop64(u, src32b, dst32b)

    abc = _tc_call(
        _tc3_body,
        [_spec3(F), _spec1(), _spec2(F), _spec2(F), _full((F,)),
         _full((3 * F, 48))],
        _spec2(48),
        jax.ShapeDtypeStruct((NPAD, 48), jnp.float32),
    )(r3, invd, h1a, qb, b1c, W2abc)

    avec = abc[:, 0:16]
    bvec = abc[:, 16:32]
    cvec = abc[:, 32:48]

    def addnorm(r, addv):
        return _tc_call(
            _tc_addnorm_body,
            [_spec3(16), _spec1(), _spec2(16)],
            _spec2(16),
            jax.ShapeDtypeStruct((NPAD, 16), jnp.float32),
        )(r, invd, addv)

    r4 = _prop16(cvec, src32, dst32)
    s1 = addnorm(r4, bvec)
    r5 = _prop16(s1, src32, dst32)
    s2 = addnorm(r5, avec)
    r6 = _prop16(s2, src32, dst32)

    out = _tc_call(
        _tc_final_body,
        [_spec3(16), _spec1(), _full((C,))],
        _spec2(C),
        jax.ShapeDtypeStruct((NPAD, C), jnp.float32),
    )(r6, invd, b2)

    return out[:N]


# speedup vs baseline: 1.0021x; 1.0021x over previous
import functools

import jax
import jax.numpy as jnp
from jax import lax
from jax.experimental import pallas as pl
from jax.experimental.pallas import tpu as pltpu
from jax.experimental.pallas import tpu_sc as plsc

N, D, E, F, C = 10000, 128, 320000, 64, 8

NC, NS = 2, 16
NW = NC * NS
K = 80
NB32 = E // NW // K
NB16 = E // NS // K
NPAD = 10240
ZCH = 64
RPS = NPAD // NS
BR = 1280

_MESH = dict(core_axis_name="c", subcore_axis_name="s", num_cores=NC,
             num_subcores=NS)
_SC_PARAMS = pltpu.CompilerParams(use_tc_tiling_on_sc=False)
NBUF = 5


def _pipelined_edges(h_ref, src_v, dst_v, rows_v, sems, acc, nb, ones_v,
                     dacc):
    nbuf = len(sems)
    for b in range(nbuf):
        pltpu.async_copy(h_ref.at[src_v.at[b]], rows_v.at[b], sems[b])

    def outer(i, _):
        for b in range(nbuf):
            j = i * nbuf + b
            pltpu.make_async_copy(
                h_ref.at[src_v.at[j]], rows_v.at[b], sems[b]).wait()
            pltpu.sync_copy(rows_v.at[b], acc.at[dst_v.at[j]], add=True)
            if ones_v is not None:
                pltpu.sync_copy(ones_v, dacc.at[dst_v.at[j]], add=True)
            nj = j + nbuf

            @pl.when(nj < nb)
            def _():
                pltpu.async_copy(h_ref.at[src_v.at[nj]], rows_v.at[b],
                                 sems[b])
        return 0

    lax.fori_loop(0, nb // nbuf, outer, 0)


def _zero_rows(buf, rows, w):
    assert w % 16 == 0, w
    c16 = w // 16
    zv = jnp.zeros((16,), jnp.float32)

    def body(i, _):
        buf[i // c16, pl.ds((i % c16) * 16, 16)] = zv
        return 0

    lax.fori_loop(0, rows * c16, body, 0)


def _make_prop_split(with_deg):
    W = 64
    mesh = plsc.VectorSubcoreMesh(**_MESH)
    if with_deg:
        out_type = (jax.ShapeDtypeStruct((NC, NPAD, W), jnp.float32),
                    jax.ShapeDtypeStruct((NPAD,), jnp.float32))
    else:
        out_type = jax.ShapeDtypeStruct((NC, NPAD, W), jnp.float32)

    scratch = [
        pltpu.VMEM((NB16, K), jnp.int32),
        pltpu.VMEM((NB16, K), jnp.int32),
        pltpu.VMEM((NBUF, K, W), jnp.float32),
        pltpu.VMEM((ZCH, W), jnp.float32),
        pltpu.VMEM_SHARED((NPAD, W), jnp.float32),
    ] + [pltpu.SemaphoreType.DMA] * NBUF
    if with_deg:
        scratch.append(pltpu.VMEM((K,), jnp.float32))
        scratch.append(pltpu.VMEM((RPS,), jnp.float32))
        scratch.append(pltpu.VMEM_SHARED((NPAD,), jnp.float32))

    def body(ha_hbm, hb_hbm, src_hbm, dst_hbm, *rest):
        if with_deg:
            (out_hbm, deg_hbm, src_v, dst_v, rows_v, zbuf, acc,
             s0, s1, s2, s3, s4, ones_v, zrow, dacc) = rest
        else:
            (out_hbm, src_v, dst_v, rows_v, zbuf, acc,
             s0, s1, s2, s3, s4) = rest
            deg_hbm = ones_v = zrow = dacc = None
        sems = (s0, s1, s2, s3, s4)
        cid = lax.axis_index("c")
        sid = lax.axis_index("s")
        lo = sid * RPS

        _zero_rows(zbuf, ZCH, W)
        for t in range(RPS // ZCH):
            pltpu.sync_copy(zbuf, acc.at[pl.ds(lo + t * ZCH, ZCH)])
        if with_deg:
            def dz(i, _):
                zrow[pl.ds(i * 16, 16)] = jnp.zeros((16,), jnp.float32)
                return 0
            lax.fori_loop(0, RPS // 16, dz, 0)

            def of(i, _):
                ones_v[pl.ds(i * 16, 16)] = jnp.ones((16,), jnp.float32)
                return 0
            lax.fori_loop(0, K // 16, of, 0)

            @pl.when(cid == 0)
            def _():
                pltpu.sync_copy(zrow, dacc.at[pl.ds(lo, RPS)])

        pltpu.sync_copy(src_hbm.at[sid], src_v)
        pltpu.sync_copy(dst_hbm.at[sid], dst_v)

        plsc.subcore_barrier()

        def edge_loop(h_ref, do_deg):
            _pipelined_edges(h_ref, src_v, dst_v, rows_v, sems, acc, NB16,
                             ones_v if do_deg else None, dacc)

        @pl.when(cid == 0)
        def _():
            edge_loop(ha_hbm, with_deg)

        @pl.when(cid == 1)
        def _():
            edge_loop(hb_hbm, False)

        plsc.subcore_barrier()

        pltpu.sync_copy(acc.at[pl.ds(lo, RPS)],
                        out_hbm.at[cid, pl.ds(lo, RPS)])
        if with_deg:
            @pl.when(cid == 0)
            def _():
                pltpu.sync_copy(dacc.at[pl.ds(lo, RPS)],
                                deg_hbm.at[pl.ds(lo, RPS)])

    return pl.kernel(body, out_type=out_type, mesh=mesh,
                     scratch_types=scratch, compiler_params=_SC_PARAMS)


def _make_prop_partial(W):
    nbuf = 25 if W <= 16 else NBUF
    mesh = plsc.VectorSubcoreMesh(**_MESH)
    scratch = [
        pltpu.VMEM((NB32, K), jnp.int32),
        pltpu.VMEM((NB32, K), jnp.int32),
        pltpu.VMEM((nbuf, K, W), jnp.float32),
        pltpu.VMEM((ZCH, W), jnp.float32),
        pltpu.VMEM_SHARED((NPAD, W), jnp.float32),
    ] + [pltpu.SemaphoreType.DMA] * nbuf

    def body(h_hbm, src_hbm, dst_hbm, out_hbm, src_v, dst_v, rows_v, zbuf,
             acc, *sems):
        cid = lax.axis_index("c")
        sid = lax.axis_index("s")
        w = cid * NS + sid

        _zero_rows(zbuf, ZCH, W)
        for t in range(RPS // ZCH):
            pltpu.sync_copy(zbuf, acc.at[pl.ds(sid * RPS + t * ZCH, ZCH)])

        pltpu.sync_copy(src_hbm.at[w], src_v)
        pltpu.sync_copy(dst_hbm.at[w], dst_v)

        plsc.subcore_barrier()

        _pipelined_edges(h_hbm, src_v, dst_v, rows_v, sems, acc, NB32,
                         None, None)

        plsc.subcore_barrier()

        pltpu.sync_copy(acc.at[pl.ds(sid * RPS, RPS)],
                        out_hbm.at[cid, pl.ds(sid * RPS, RPS)])

    return pl.kernel(
        body,
        out_type=jax.ShapeDtypeStruct((NC, NPAD, W), jnp.float32),
        mesh=mesh,
        scratch_types=scratch,
        compiler_params=_SC_PARAMS,
    )


@functools.lru_cache(maxsize=None)
def _get_prop(kind):
    if kind == "split_deg":
        return _make_prop_split(True)
    if kind == "split":
        return _make_prop_split(False)
    return _make_prop_partial(kind)


def _prop128_deg(*a):
    return _get_prop("split_deg")(*a)


def _prop128(*a):
    return _get_prop("split")(*a)


def _prop64(*a):
    return _get_prop(64)(*a)


def _prop16(*a):
    return _get_prop(16)(*a)


def _tc_call(body_fn, in_specs, out_specs, out_shapes):
    return pl.pallas_call(
        body_fn,
        grid=(NPAD // BR,),
        in_specs=in_specs,
        out_specs=out_specs,
        out_shape=out_shapes,
    )


def _spec3(w):
    return pl.BlockSpec((NC, BR, w), lambda i: (0, i, 0))


def _spec2(w):
    return pl.BlockSpec((BR, w), lambda i: (i, 0))


def _spec1():
    return pl.BlockSpec((BR, 1), lambda i: (i, 0))


def _full(shape):
    nd = len(shape)
    return pl.BlockSpec(shape, lambda i: (0,) * nd)


def _tc1_body(p_ref, dg_ref, w1a_ref, w1bc_ref, b1a_ref,
              z0_ref, z1_ref, h1a_ref, invd_ref):
    inv = 1.0 / jnp.maximum(dg_ref[...], 1.0)
    h1 = jnp.concatenate([p_ref[0], p_ref[1]], axis=1) * inv
    z = jnp.dot(h1, w1bc_ref[...], preferred_element_type=jnp.float32)
    z0_ref[...] = z[:, :F]
    z1_ref[...] = z[:, F:]
    h1a_ref[...] = (jnp.dot(h1, w1a_ref[...], preferred_element_type=jnp.float32)
                    + b1a_ref[...][None, :])
    invd_ref[...] = inv


def _tc2_body(r_ref, invd_ref, b1b_ref, u_ref, qb_ref):
    inv = invd_ref[...]
    u_ref[...] = r_ref[1] * inv
    qb_ref[...] = r_ref[0] * inv + b1b_ref[...][None, :]


def _tc2_body(r_ref, invd_ref, b1b_ref, u_ref, qb_ref):
    inv = invd_ref[...]
    u_ref[...] = r_ref[1] * inv
    qb_ref[...] = r_ref[0] * inv + b1b_ref[...][None, :]


def _tc3_body(r_ref, invd_ref, h1a_ref, qb_ref, b1c_ref, w2_ref, abc_ref):
    v = (r_ref[0] + r_ref[1]) * invd_ref[...] + b1c_ref[...][None, :]
    h = jax.nn.relu(jnp.concatenate([h1a_ref[...], qb_ref[...], v], axis=1))
    abc_ref[...] = jnp.dot(h, w2_ref[...], preferred_element_type=jnp.float32)


def _tc_addnorm_body(r_ref, invd_ref, add_ref, o_ref):
    o_ref[...] = add_ref[...] + (r_ref[0] + r_ref[1]) * invd_ref[...]


def _tc_final_body(r_ref, invd_ref, b2_ref, o_ref):
    t = (r_ref[0] + r_ref[1])[:, :C] * invd_ref[...]
    o_ref[...] = jax.nn.relu(t + b2_ref[...][None, :])


def kernel(x, edge_index, W1a, W1b, W1c, b1a, b1b, b1c, W2a, W2b, W2c, b2):
    src16 = edge_index[0].reshape(NS, NB16, K)
    dst16 = edge_index[1].reshape(NS, NB16, K)
    src32 = edge_index[0].reshape(NW, NB32, K)
    dst32 = edge_index[1].reshape(NW, NB32, K)
    W1bc = jnp.concatenate([W1b, W1c], axis=1)
    zc = jnp.zeros((3 * F, 8), jnp.float32)
    W2abc = jnp.concatenate([W2a, zc, W2b, zc, W2c, zc], axis=1)

    p1, deg = _prop128_deg(x[:, :F], x[:, F:], src16, dst16)
    deg2 = deg.reshape(NPAD, 1)

    z0, z1, h1a, invd = _tc_call(
        _tc1_body,
        [_spec3(F), _spec1(), _full((D, F)), _full((D, D)), _full((F,))],
        [_spec2(F), _spec2(F), _spec2(F), _spec1()],
        [jax.ShapeDtypeStruct((NPAD, F), jnp.float32),
         jax.ShapeDtypeStruct((NPAD, F), jnp.float32),
         jax.ShapeDtypeStruct((NPAD, F), jnp.float32),
         jax.ShapeDtypeStruct((NPAD, 1), jnp.float32)],
    )(p1, deg2, W1a, W1bc, b1a)

    r2 = _prop128(z0, z1, src16, dst16)

    u, qb = _tc_call(
        _tc2_body,
        [_spec3(F), _spec1(), _full((F,))],
        [_spec2(F), _spec2(F)],
        [jax.ShapeDtypeStruct((NPAD, F), jnp.float32),
         jax.ShapeDtypeStruct((NPAD, F), jnp.float32)],
    )(r2, invd, b1b)

    r3 = _prop64(u, src32, dst32)

    abc = _tc_call(
        _tc3_body,
        [_spec3(F), _spec1(), _spec2(F), _spec2(F), _full((F,)),
         _full((3 * F, 48))],
        _spec2(48),
        jax.ShapeDtypeStruct((NPAD, 48), jnp.float32),
    )(r3, invd, h1a, qb, b1c, W2abc)

    avec = abc[:, 0:16]
    bvec = abc[:, 16:32]
    cvec = abc[:, 32:48]

    def addnorm(r, addv):
        return _tc_call(
            _tc_addnorm_body,
            [_spec3(16), _spec1(), _spec2(16)],
            _spec2(16),
            jax.ShapeDtypeStruct((NPAD, 16), jnp.float32),
        )(r, invd, addv)

    r4 = _prop16(cvec, src32, dst32)
    s1 = addnorm(r4, bvec)
    r5 = _prop16(s1, src32, dst32)
    s2 = addnorm(r5, avec)
    r6 = _prop16(s2, src32, dst32)

    out = _tc_call(
        _tc_final_body,
        [_spec3(16), _spec1(), _full((C,))],
        _spec2(C),
        jax.ShapeDtypeStruct((NPAD, C), jnp.float32),
    )(r6, invd, b2)

    return out[:N]
